# baseline (speedup 1.0000x reference)
import functools

import jax
import jax.numpy as jnp
from jax.experimental import pallas as pl
from jax.experimental.pallas import tpu as pltpu

E = 64
TOPK = 2
S_TILE = 512


def _router_kernel(x_ref, wr_ref, br_ref, idx_ref, w_ref):
    xm = jnp.mean(x_ref[0], axis=0, keepdims=True)
    logits = jnp.dot(xm, wr_ref[...], preferred_element_type=jnp.float32)
    logits = logits + br_ref[...]
    m = jnp.max(logits, axis=-1, keepdims=True)
    p = jnp.exp(logits - m)
    p = p / jnp.sum(p, axis=-1, keepdims=True)
    ids = jax.lax.broadcasted_iota(jnp.int32, p.shape, 1)
    big = jnp.int32(E)
    m1 = jnp.max(p, axis=-1, keepdims=True)
    i1 = jnp.min(jnp.where(p == m1, ids, big), axis=-1, keepdims=True)
    p2 = jnp.where(ids == i1, -1.0, p)
    m2 = jnp.max(p2, axis=-1, keepdims=True)
    i2 = jnp.min(jnp.where(p2 == m2, ids, big), axis=-1, keepdims=True)
    t = jnp.exp(m2 - m1)
    w1 = 1.0 / (1.0 + t)
    w2 = t / (1.0 + t)
    lane = jax.lax.broadcasted_iota(jnp.int32, (1, TOPK), 1)
    idx_ref[0] = jnp.where(lane == 0, i1, i2).astype(jnp.int32)
    w_ref[0] = jnp.where(lane == 0, w1, w2)


def _moe_kernel(idx_sref, w_sref, x_ref, w1_ref, w2_ref, b1_ref, b2_ref,
                gamma_ref, beta_ref, out_ref):
    b = pl.program_id(0)
    k = pl.program_id(2)
    xb = x_ref[0]
    h = jnp.dot(xb.astype(jnp.bfloat16), w1_ref[0].astype(jnp.bfloat16),
                preferred_element_type=jnp.float32)
    h = jnp.maximum(h + b1_ref[0], 0.0)
    o = jnp.dot(h.astype(jnp.bfloat16), w2_ref[0].astype(jnp.bfloat16),
                preferred_element_type=jnp.float32)
    o = o + b2_ref[0]
    o = o * w_sref[b * TOPK + k]

    @pl.when(k == 0)
    def _init():
        out_ref[0] = o

    @pl.when(k == TOPK - 1)
    def _finish():
        y = out_ref[0] + o + xb
        mu = jnp.mean(y, axis=-1, keepdims=True)
        yc = y - mu
        var = jnp.mean(yc * yc, axis=-1, keepdims=True)
        out_ref[0] = yc * jax.lax.rsqrt(var + 1e-5) * gamma_ref[...] + beta_ref[...]


@jax.jit
def kernel(x, Wr, br, W1, b1, W2, b2, gamma, beta):
    B, S, D = x.shape
    F = W1.shape[2]

    idx, w = pl.pallas_call(
        _router_kernel,
        grid=(B,),
        in_specs=[
            pl.BlockSpec((1, S, D), lambda b: (b, 0, 0)),
            pl.BlockSpec((D, E), lambda b: (0, 0)),
            pl.BlockSpec((1, E), lambda b: (0, 0)),
        ],
        out_specs=[
            pl.BlockSpec((1, 1, TOPK), lambda b: (b, 0, 0)),
            pl.BlockSpec((1, 1, TOPK), lambda b: (b, 0, 0)),
        ],
        out_shape=[
            jax.ShapeDtypeStruct((B, 1, TOPK), jnp.int32),
            jax.ShapeDtypeStruct((B, 1, TOPK), jnp.float32),
        ],
    )(x, Wr, br.reshape(1, E))

    idx_flat = idx.reshape(-1)
    w_flat = w.reshape(-1)

    n_s = S // S_TILE
    out = pl.pallas_call(
        _moe_kernel,
        grid_spec=pltpu.PrefetchScalarGridSpec(
            num_scalar_prefetch=2,
            grid=(B, n_s, TOPK),
            in_specs=[
                pl.BlockSpec((1, S_TILE, D), lambda b, s, k, idx, w: (b, s, 0)),
                pl.BlockSpec((1, D, F), lambda b, s, k, idx, w: (idx[b * TOPK + k], 0, 0)),
                pl.BlockSpec((1, F, D), lambda b, s, k, idx, w: (idx[b * TOPK + k], 0, 0)),
                pl.BlockSpec((1, 1, F), lambda b, s, k, idx, w: (idx[b * TOPK + k], 0, 0)),
                pl.BlockSpec((1, 1, D), lambda b, s, k, idx, w: (idx[b * TOPK + k], 0, 0)),
                pl.BlockSpec((D,), lambda b, s, k, idx, w: (0,)),
                pl.BlockSpec((D,), lambda b, s, k, idx, w: (0,)),
            ],
            out_specs=pl.BlockSpec((1, S_TILE, D), lambda b, s, k, idx, w: (b, s, 0)),
        ),
        out_shape=jax.ShapeDtypeStruct((B, S, D), jnp.float32),
    )(idx_flat, w_flat, x, W1, W2, b1.reshape(E, 1, F), b2.reshape(E, 1, D),
      gamma, beta)

    return out


# speedup vs baseline: 1.2710x; 1.2710x over previous
import functools

import jax
import jax.numpy as jnp
from jax.experimental import pallas as pl
from jax.experimental.pallas import tpu as pltpu

E = 64
TOPK = 2
S_TILE = 512


def _router_kernel(x_ref, wr_ref, br_ref, idx_ref, w_ref):
    xm = jnp.mean(x_ref[0], axis=0, keepdims=True)
    logits = jnp.dot(xm, wr_ref[...], preferred_element_type=jnp.float32)
    logits = logits + br_ref[...]
    m = jnp.max(logits, axis=-1, keepdims=True)
    p = jnp.exp(logits - m)
    p = p / jnp.sum(p, axis=-1, keepdims=True)
    ids = jax.lax.broadcasted_iota(jnp.int32, p.shape, 1)
    big = jnp.int32(E)
    m1 = jnp.max(p, axis=-1, keepdims=True)
    i1 = jnp.min(jnp.where(p == m1, ids, big), axis=-1, keepdims=True)
    p2 = jnp.where(ids == i1, -1.0, p)
    m2 = jnp.max(p2, axis=-1, keepdims=True)
    i2 = jnp.min(jnp.where(p2 == m2, ids, big), axis=-1, keepdims=True)
    t = jnp.exp(m2 - m1)
    w1 = 1.0 / (1.0 + t)
    w2 = t / (1.0 + t)
    lane = jax.lax.broadcasted_iota(jnp.int32, (1, TOPK), 1)
    idx_ref[0] = jnp.where(lane == 0, i1, i2).astype(jnp.int32)
    w_ref[0] = jnp.where(lane == 0, w1, w2)


def _moe_kernel(idx_sref, w_sref, x_ref, w1a_ref, w1b_ref, w2a_ref, w2b_ref,
                b1a_ref, b1b_ref, b2a_ref, b2b_ref, gamma_ref, beta_ref,
                out_ref):
    b = pl.program_id(0)
    xb = x_ref[0]
    ha = jnp.maximum(
        jnp.dot(xb, w1a_ref[0], preferred_element_type=jnp.float32)
        + b1a_ref[0], 0.0)
    oa = jnp.dot(ha, w2a_ref[0], preferred_element_type=jnp.float32)
    oa = (oa + b2a_ref[0]) * w_sref[b * TOPK]
    hb = jnp.maximum(
        jnp.dot(xb, w1b_ref[0], preferred_element_type=jnp.float32)
        + b1b_ref[0], 0.0)
    ob = jnp.dot(hb, w2b_ref[0], preferred_element_type=jnp.float32)
    ob = (ob + b2b_ref[0]) * w_sref[b * TOPK + 1]
    y = oa + ob + xb
    mu = jnp.mean(y, axis=-1, keepdims=True)
    yc = y - mu
    var = jnp.mean(yc * yc, axis=-1, keepdims=True)
    out_ref[0] = yc * jax.lax.rsqrt(var + 1e-5) * gamma_ref[...] + beta_ref[...]


@jax.jit
def kernel(x, Wr, br, W1, b1, W2, b2, gamma, beta):
    B, S, D = x.shape
    F = W1.shape[2]

    idx, w = pl.pallas_call(
        _router_kernel,
        grid=(B,),
        in_specs=[
            pl.BlockSpec((1, S, D), lambda b: (b, 0, 0)),
            pl.BlockSpec((D, E), lambda b: (0, 0)),
            pl.BlockSpec((1, E), lambda b: (0, 0)),
        ],
        out_specs=[
            pl.BlockSpec((1, 1, TOPK), lambda b: (b, 0, 0)),
            pl.BlockSpec((1, 1, TOPK), lambda b: (b, 0, 0)),
        ],
        out_shape=[
            jax.ShapeDtypeStruct((B, 1, TOPK), jnp.int32),
            jax.ShapeDtypeStruct((B, 1, TOPK), jnp.float32),
        ],
    )(x, Wr, br.reshape(1, E))

    idx_flat = idx.reshape(-1)
    w_flat = w.reshape(-1)
    b1r = b1.reshape(E, 1, F)
    b2r = b2.reshape(E, 1, D)

    n_s = S // S_TILE
    out = pl.pallas_call(
        _moe_kernel,
        grid_spec=pltpu.PrefetchScalarGridSpec(
            num_scalar_prefetch=2,
            grid=(B, n_s),
            in_specs=[
                pl.BlockSpec((1, S_TILE, D), lambda b, s, idx, w: (b, s, 0)),
                pl.BlockSpec((1, D, F), lambda b, s, idx, w: (idx[b * TOPK], 0, 0)),
                pl.BlockSpec((1, D, F), lambda b, s, idx, w: (idx[b * TOPK + 1], 0, 0)),
                pl.BlockSpec((1, F, D), lambda b, s, idx, w: (idx[b * TOPK], 0, 0)),
                pl.BlockSpec((1, F, D), lambda b, s, idx, w: (idx[b * TOPK + 1], 0, 0)),
                pl.BlockSpec((1, 1, F), lambda b, s, idx, w: (idx[b * TOPK], 0, 0)),
                pl.BlockSpec((1, 1, F), lambda b, s, idx, w: (idx[b * TOPK + 1], 0, 0)),
                pl.BlockSpec((1, 1, D), lambda b, s, idx, w: (idx[b * TOPK], 0, 0)),
                pl.BlockSpec((1, 1, D), lambda b, s, idx, w: (idx[b * TOPK + 1], 0, 0)),
                pl.BlockSpec((D,), lambda b, s, idx, w: (0,)),
                pl.BlockSpec((D,), lambda b, s, idx, w: (0,)),
            ],
            out_specs=pl.BlockSpec((1, S_TILE, D), lambda b, s, idx, w: (b, s, 0)),
        ),
        out_shape=jax.ShapeDtypeStruct((B, S, D), jnp.float32),
    )(idx_flat, w_flat, x, W1, W1, W2, W2, b1r, b1r, b2r, b2r, gamma, beta)

    return out


# speedup vs baseline: 1.3769x; 1.0833x over previous
import functools

import jax
import jax.numpy as jnp
from jax.experimental import pallas as pl
from jax.experimental.pallas import tpu as pltpu

E = 64
TOPK = 2
S_TILE = 1024


def _router_kernel(x_ref, wr_ref, br_ref, idx_ref, w_ref):
    xm = jnp.mean(x_ref[0], axis=0, keepdims=True)
    logits = jnp.dot(xm, wr_ref[...], preferred_element_type=jnp.float32)
    logits = logits + br_ref[...]
    m = jnp.max(logits, axis=-1, keepdims=True)
    p = jnp.exp(logits - m)
    p = p / jnp.sum(p, axis=-1, keepdims=True)
    ids = jax.lax.broadcasted_iota(jnp.int32, p.shape, 1)
    big = jnp.int32(E)
    m1 = jnp.max(p, axis=-1, keepdims=True)
    i1 = jnp.min(jnp.where(p == m1, ids, big), axis=-1, keepdims=True)
    p2 = jnp.where(ids == i1, -1.0, p)
    m2 = jnp.max(p2, axis=-1, keepdims=True)
    i2 = jnp.min(jnp.where(p2 == m2, ids, big), axis=-1, keepdims=True)
    t = jnp.exp(m2 - m1)
    w1 = 1.0 / (1.0 + t)
    w2 = t / (1.0 + t)
    lane = jax.lax.broadcasted_iota(jnp.int32, (1, TOPK), 1)
    idx_ref[0] = jnp.where(lane == 0, i1, i2).astype(jnp.int32)
    w_ref[0] = jnp.where(lane == 0, w1, w2)


def _moe_kernel(idx_sref, w_sref, x_ref, w1a_ref, w1b_ref, w2a_ref, w2b_ref,
                b1a_ref, b1b_ref, b2a_ref, b2b_ref, gamma_ref, beta_ref,
                out_ref):
    b = pl.program_id(0)
    xb = x_ref[0]
    ha = jnp.maximum(
        jnp.dot(xb, w1a_ref[0], preferred_element_type=jnp.float32)
        + b1a_ref[0], 0.0)
    oa = jnp.dot(ha, w2a_ref[0], preferred_element_type=jnp.float32)
    oa = (oa + b2a_ref[0]) * w_sref[b * TOPK]
    hb = jnp.maximum(
        jnp.dot(xb, w1b_ref[0], preferred_element_type=jnp.float32)
        + b1b_ref[0], 0.0)
    ob = jnp.dot(hb, w2b_ref[0], preferred_element_type=jnp.float32)
    ob = (ob + b2b_ref[0]) * w_sref[b * TOPK + 1]
    y = oa + ob + xb
    mu = jnp.mean(y, axis=-1, keepdims=True)
    yc = y - mu
    var = jnp.mean(yc * yc, axis=-1, keepdims=True)
    out_ref[0] = yc * jax.lax.rsqrt(var + 1e-5) * gamma_ref[...] + beta_ref[...]


@jax.jit
def kernel(x, Wr, br, W1, b1, W2, b2, gamma, beta):
    B, S, D = x.shape
    F = W1.shape[2]

    idx, w = pl.pallas_call(
        _router_kernel,
        grid=(B,),
        in_specs=[
            pl.BlockSpec((1, S, D), lambda b: (b, 0, 0)),
            pl.BlockSpec((D, E), lambda b: (0, 0)),
            pl.BlockSpec((1, E), lambda b: (0, 0)),
        ],
        out_specs=[
            pl.BlockSpec((1, 1, TOPK), lambda b: (b, 0, 0)),
            pl.BlockSpec((1, 1, TOPK), lambda b: (b, 0, 0)),
        ],
        out_shape=[
            jax.ShapeDtypeStruct((B, 1, TOPK), jnp.int32),
            jax.ShapeDtypeStruct((B, 1, TOPK), jnp.float32),
        ],
    )(x, Wr, br.reshape(1, E))

    idx_flat = idx.reshape(-1)
    w_flat = w.reshape(-1)
    b1r = b1.reshape(E, 1, F)
    b2r = b2.reshape(E, 1, D)

    n_s = S // S_TILE
    out = pl.pallas_call(
        _moe_kernel,
        grid_spec=pltpu.PrefetchScalarGridSpec(
            num_scalar_prefetch=2,
            grid=(B, n_s),
            in_specs=[
                pl.BlockSpec((1, S_TILE, D), lambda b, s, idx, w: (b, s, 0)),
                pl.BlockSpec((1, D, F), lambda b, s, idx, w: (idx[b * TOPK], 0, 0)),
                pl.BlockSpec((1, D, F), lambda b, s, idx, w: (idx[b * TOPK + 1], 0, 0)),
                pl.BlockSpec((1, F, D), lambda b, s, idx, w: (idx[b * TOPK], 0, 0)),
                pl.BlockSpec((1, F, D), lambda b, s, idx, w: (idx[b * TOPK + 1], 0, 0)),
                pl.BlockSpec((1, 1, F), lambda b, s, idx, w: (idx[b * TOPK], 0, 0)),
                pl.BlockSpec((1, 1, F), lambda b, s, idx, w: (idx[b * TOPK + 1], 0, 0)),
                pl.BlockSpec((1, 1, D), lambda b, s, idx, w: (idx[b * TOPK], 0, 0)),
                pl.BlockSpec((1, 1, D), lambda b, s, idx, w: (idx[b * TOPK + 1], 0, 0)),
                pl.BlockSpec((D,), lambda b, s, idx, w: (0,)),
                pl.BlockSpec((D,), lambda b, s, idx, w: (0,)),
            ],
            out_specs=pl.BlockSpec((1, S_TILE, D), lambda b, s, idx, w: (b, s, 0)),
        ),
        out_shape=jax.ShapeDtypeStruct((B, S, D), jnp.float32),
    )(idx_flat, w_flat, x, W1, W1, W2, W2, b1r, b1r, b2r, b2r, gamma, beta)

    return out
